# baseline (speedup 1.0000x reference)
import functools

import jax
import jax.numpy as jnp
from jax import lax
from jax.experimental import pallas as pl
from jax.experimental.pallas import tpu as pltpu
from jax.experimental.pallas import tpu_sc as plsc

NUM_EXPERTS = 64
K_TOP = 8
HIDDEN = 4096
TOKENS = 16384

_BT = 512
_CHAIN = (16384,)

_NC = 2
_NS = 16
_L = 16
_NW = _NC * _NS


def _scores_body(x_ref, w_ref, o_ref):
    logits = lax.dot_general(
        x_ref[...], w_ref[...],
        (((1,), (1,)), ((), ())),
        preferred_element_type=jnp.float32,
    )
    m = jnp.max(logits, axis=1, keepdims=True)
    p = jnp.exp(logits - m)
    o_ref[...] = (p / jnp.sum(p, axis=1, keepdims=True)).T


def _topk_group(sc_v, idx_v, w_v, g):
    span = pl.ds(g * _L, _L)
    keys = []
    for e in range(NUM_EXPERTS):
        b = plsc.bitcast(sc_v[e, span], jnp.uint32)
        keys.append((b & jnp.uint32(0xFFFFFFC0)) | jnp.uint32(63 - e))

    def ce(a, i, j):
        hi = jnp.maximum(a[i], a[j])
        lo = jnp.minimum(a[i], a[j])
        a[i], a[j] = hi, lo

    s8 = ((0, 1), (2, 3), (4, 5), (6, 7),
          (0, 2), (1, 3), (4, 6), (5, 7),
          (1, 2), (5, 6), (0, 4), (3, 7),
          (1, 5), (2, 6), (1, 4), (3, 6),
          (2, 4), (3, 5), (3, 4))
    blocks = []
    for blk in range(NUM_EXPERTS // 8):
        a = keys[8 * blk:8 * blk + 8]
        for i, j in s8:
            ce(a, i, j)
        blocks.append(a)
    bit12 = ((0, 4), (1, 5), (2, 6), (3, 7),
             (0, 2), (1, 3), (4, 6), (5, 7),
             (0, 1), (2, 3), (4, 5), (6, 7))
    while len(blocks) > 1:
        nxt = []
        for p in range(0, len(blocks), 2):
            a, b = blocks[p], blocks[p + 1]
            c = [jnp.maximum(a[i], b[7 - i]) for i in range(8)]
            for i, j in bit12:
                ce(c, i, j)
            nxt.append(c)
        blocks = nxt
    top = blocks[0]
    tokloc = lax.iota(jnp.int32, _L) + g * _L
    pairs = []
    for j in range(K_TOP):
        ij = jnp.int32(63) - plsc.bitcast(top[j] & jnp.uint32(63), jnp.int32)
        wj = plsc.load_gather(sc_v, [ij, tokloc])
        pairs.append((ij, wj))
    for i, j in s8:
        ia, wa = pairs[i]
        ib, wb = pairs[j]
        ka = plsc.bitcast(wa, jnp.uint32)
        kb = plsc.bitcast(wb, jnp.uint32)
        sw = (kb > ka) | ((kb == ka) & (ib < ia))
        pairs[i] = (jnp.where(sw, ib, ia), jnp.where(sw, wb, wa))
        pairs[j] = (jnp.where(sw, ia, ib), jnp.where(sw, wa, wb))
    for j in range(K_TOP):
        idx_v[j, span] = pairs[j][0]
        w_v[j, span] = pairs[j][1]


def _make_sc_topk(ntok):
    chunk = ntok // _NW
    ngroups = chunk // _L

    @functools.partial(
        pl.kernel,
        out_type=(
            jax.ShapeDtypeStruct((K_TOP, ntok), jnp.int32),
            jax.ShapeDtypeStruct((K_TOP, ntok), jnp.float32),
        ),
        mesh=plsc.VectorSubcoreMesh(core_axis_name="c", subcore_axis_name="s"),
        compiler_params=pltpu.CompilerParams(
            needs_layout_passes=False, use_tc_tiling_on_sc=False),
        scratch_types=[
            pltpu.VMEM((NUM_EXPERTS, chunk), jnp.float32),
            pltpu.VMEM((K_TOP, chunk), jnp.int32),
            pltpu.VMEM((K_TOP, chunk), jnp.float32),
        ],
    )
    def sc_topk(scores_hbm, idx_hbm, w_hbm, sc_v, idx_v, w_v):
        wid = lax.axis_index("s") * _NC + lax.axis_index("c")
        base = wid * chunk
        pltpu.sync_copy(scores_hbm.at[:, pl.ds(base, chunk)], sc_v)

        def group2(g, carry):
            _topk_group(sc_v, idx_v, w_v, 2 * g)
            _topk_group(sc_v, idx_v, w_v, 2 * g + 1)
            return carry

        lax.fori_loop(0, ngroups // 2, group2, 0)
        pltpu.sync_copy(idx_v, idx_hbm.at[:, pl.ds(base, chunk)])
        pltpu.sync_copy(w_v, w_hbm.at[:, pl.ds(base, chunk)])

    return sc_topk


_sc_topk_by_size = {n: _make_sc_topk(n) for n in set(_CHAIN)}


def _make_scores_call(block_off, ntok):
    nsteps = ntok // _BT

    return pl.pallas_call(
        _scores_body,
        grid=(nsteps,),
        in_specs=[
            pl.BlockSpec((_BT, HIDDEN), lambda i, o=block_off: (o + i, 0)),
            pl.BlockSpec((NUM_EXPERTS, HIDDEN), lambda i: (0, 0)),
        ],
        out_specs=pl.BlockSpec((NUM_EXPERTS, _BT), lambda i: (0, i)),
        out_shape=jax.ShapeDtypeStruct((NUM_EXPERTS, ntok), jnp.float32),
    )


def kernel(hidden_states, weight):
    idxs, ws = [], []
    off = 0
    for ntok in _CHAIN:
        scores_c = _make_scores_call(off // _BT, ntok)(hidden_states, weight)
        i_c, w_c = _sc_topk_by_size[ntok](scores_c)
        idxs.append(i_c)
        ws.append(w_c)
        off += ntok
    idx_t = jnp.concatenate(idxs, axis=1)
    w_t = jnp.concatenate(ws, axis=1)
    return idx_t.T, w_t.T


# speedup vs baseline: 1.0106x; 1.0106x over previous
import functools

import jax
import jax.numpy as jnp
from jax import lax
from jax.experimental import pallas as pl
from jax.experimental.pallas import tpu as pltpu
from jax.experimental.pallas import tpu_sc as plsc

NUM_EXPERTS = 64
K_TOP = 8
HIDDEN = 4096
TOKENS = 16384

_BT = 1024
_CHAIN = (16384,)

_NC = 2
_NS = 16
_L = 16
_NW = _NC * _NS


def _scores_body(x_ref, w_ref, o_ref):
    logits = lax.dot_general(
        x_ref[...], w_ref[...],
        (((1,), (1,)), ((), ())),
        preferred_element_type=jnp.float32,
    )
    m = jnp.max(logits, axis=1, keepdims=True)
    p = jnp.exp(logits - m)
    o_ref[...] = (p / jnp.sum(p, axis=1, keepdims=True)).T


def _topk_group(sc_v, idx_v, w_v, g):
    span = pl.ds(g * _L, _L)
    keys = []
    for e in range(NUM_EXPERTS):
        b = plsc.bitcast(sc_v[e, span], jnp.uint32)
        keys.append((b & jnp.uint32(0xFFFFFFC0)) | jnp.uint32(63 - e))

    def ce(a, i, j):
        hi = jnp.maximum(a[i], a[j])
        lo = jnp.minimum(a[i], a[j])
        a[i], a[j] = hi, lo

    s8 = ((0, 1), (2, 3), (4, 5), (6, 7),
          (0, 2), (1, 3), (4, 6), (5, 7),
          (1, 2), (5, 6), (0, 4), (3, 7),
          (1, 5), (2, 6), (1, 4), (3, 6),
          (2, 4), (3, 5), (3, 4))
    blocks = []
    for blk in range(NUM_EXPERTS // 8):
        a = keys[8 * blk:8 * blk + 8]
        for i, j in s8:
            ce(a, i, j)
        blocks.append(a)
    bit12 = ((0, 4), (1, 5), (2, 6), (3, 7),
             (0, 2), (1, 3), (4, 6), (5, 7),
             (0, 1), (2, 3), (4, 5), (6, 7))
    while len(blocks) > 1:
        nxt = []
        for p in range(0, len(blocks), 2):
            a, b = blocks[p], blocks[p + 1]
            c = [jnp.maximum(a[i], b[7 - i]) for i in range(8)]
            for i, j in bit12:
                ce(c, i, j)
            nxt.append(c)
        blocks = nxt
    top = blocks[0]
    tokloc = lax.iota(jnp.int32, _L) + g * _L
    pairs = []
    for j in range(K_TOP):
        ij = jnp.int32(63) - plsc.bitcast(top[j] & jnp.uint32(63), jnp.int32)
        wj = plsc.load_gather(sc_v, [ij, tokloc])
        pairs.append((ij, wj))
    for i, j in s8:
        ia, wa = pairs[i]
        ib, wb = pairs[j]
        ka = plsc.bitcast(wa, jnp.uint32)
        kb = plsc.bitcast(wb, jnp.uint32)
        sw = (kb > ka) | ((kb == ka) & (ib < ia))
        pairs[i] = (jnp.where(sw, ib, ia), jnp.where(sw, wb, wa))
        pairs[j] = (jnp.where(sw, ia, ib), jnp.where(sw, wa, wb))
    for j in range(K_TOP):
        idx_v[j, span] = pairs[j][0]
        w_v[j, span] = pairs[j][1]


def _make_sc_topk(ntok):
    chunk = ntok // _NW
    ngroups = chunk // _L

    @functools.partial(
        pl.kernel,
        out_type=(
            jax.ShapeDtypeStruct((K_TOP, ntok), jnp.int32),
            jax.ShapeDtypeStruct((K_TOP, ntok), jnp.float32),
        ),
        mesh=plsc.VectorSubcoreMesh(core_axis_name="c", subcore_axis_name="s"),
        compiler_params=pltpu.CompilerParams(
            needs_layout_passes=False, use_tc_tiling_on_sc=False),
        scratch_types=[
            pltpu.VMEM((NUM_EXPERTS, chunk), jnp.float32),
            pltpu.VMEM((K_TOP, chunk), jnp.int32),
            pltpu.VMEM((K_TOP, chunk), jnp.float32),
        ],
    )
    def sc_topk(scores_hbm, idx_hbm, w_hbm, sc_v, idx_v, w_v):
        wid = lax.axis_index("s") * _NC + lax.axis_index("c")
        base = wid * chunk
        pltpu.sync_copy(scores_hbm.at[:, pl.ds(base, chunk)], sc_v)

        def group2(g, carry):
            _topk_group(sc_v, idx_v, w_v, 2 * g)
            _topk_group(sc_v, idx_v, w_v, 2 * g + 1)
            return carry

        lax.fori_loop(0, ngroups // 2, group2, 0)
        pltpu.sync_copy(idx_v, idx_hbm.at[:, pl.ds(base, chunk)])
        pltpu.sync_copy(w_v, w_hbm.at[:, pl.ds(base, chunk)])

    return sc_topk


_sc_topk_by_size = {n: _make_sc_topk(n) for n in set(_CHAIN)}


def _make_scores_call(block_off, ntok):
    nsteps = ntok // _BT

    return pl.pallas_call(
        _scores_body,
        grid=(nsteps,),
        in_specs=[
            pl.BlockSpec((_BT, HIDDEN), lambda i, o=block_off: (o + i, 0)),
            pl.BlockSpec((NUM_EXPERTS, HIDDEN), lambda i: (0, 0)),
        ],
        out_specs=pl.BlockSpec((NUM_EXPERTS, _BT), lambda i: (0, i)),
        out_shape=jax.ShapeDtypeStruct((NUM_EXPERTS, ntok), jnp.float32),
    )


def kernel(hidden_states, weight):
    idxs, ws = [], []
    off = 0
    for ntok in _CHAIN:
        scores_c = _make_scores_call(off // _BT, ntok)(hidden_states, weight)
        i_c, w_c = _sc_topk_by_size[ntok](scores_c)
        idxs.append(i_c)
        ws.append(w_c)
        off += ntok
    idx_t = jnp.concatenate(idxs, axis=1)
    w_t = jnp.concatenate(ws, axis=1)
    return idx_t.T, w_t.T
